# baseline (speedup 1.0000x reference)
import functools

import jax
import jax.numpy as jnp
from jax import lax
from jax.experimental import pallas as pl
from jax.experimental.pallas import tpu as pltpu
from jax.experimental.pallas import tpu_sc as plsc

B = 16384
K = 32
NC = 2
NS = 16
NW = NC * NS
BPW = B // NW
L = 16


_mesh = plsc.VectorSubcoreMesh(core_axis_name="c", subcore_axis_name="s")


@functools.partial(
    pl.kernel,
    mesh=_mesh,
    compiler_params=pltpu.CompilerParams(
        needs_layout_passes=False, use_tc_tiling_on_sc=False
    ),
    out_type=jax.ShapeDtypeStruct((B,), jnp.float32),
    scratch_types=[
        pltpu.VMEM((BPW,), jnp.int32),
        pltpu.VMEM((BPW,), jnp.int32),
        pltpu.VMEM((K, BPW), jnp.float32),
        pltpu.VMEM((K, BPW), jnp.float32),
        pltpu.VMEM((BPW,), jnp.float32),
        pltpu.SemaphoreType.DMA,
    ],
)
def _pmf_sc(user_hbm, item_hbm, wu_t_hbm, wi_t_hbm, out_hbm,
            uidx, iidx, uels, iels, oacc, sem):
    wid = lax.axis_index("s") * NC + lax.axis_index("c")
    base = wid * BPW

    pltpu.sync_copy(user_hbm.at[pl.ds(base, BPW)], uidx)
    pltpu.sync_copy(item_hbm.at[pl.ds(base, BPW)], iidx)

    copies = []
    for k in range(K):
        copies.append(pltpu.async_copy(wu_t_hbm.at[k].at[uidx], uels.at[k], sem))
        copies.append(pltpu.async_copy(wi_t_hbm.at[k].at[iidx], iels.at[k], sem))
    for cp in copies:
        cp.wait()

    def group(g, carry):
        sl = pl.ds(g * L, L)
        acc = uels[0, sl] * iels[0, sl]
        for k in range(1, K):
            acc = acc + uels[k, sl] * iels[k, sl]
        oacc[sl] = acc
        return carry

    lax.fori_loop(0, BPW // L, group, 0)

    pltpu.sync_copy(oacc, out_hbm.at[pl.ds(base, BPW)])


def kernel(user, item, W_user, W_item):
    return _pmf_sc(user, item, W_user.T, W_item.T)


# speedup vs baseline: 1.0000x; 1.0000x over previous
import functools

import jax
import jax.numpy as jnp
from jax import lax
from jax.experimental import pallas as pl
from jax.experimental.pallas import tpu as pltpu
from jax.experimental.pallas import tpu_sc as plsc

B = 16384
K = 32
N_ROWS = 1000000
NC = 2
NS = 16
NW = NC * NS
BPW = B // NW
L = 16
W = 16
RPK = N_ROWS // W
C = 32
RPC = C * K
NCH = BPW // C
NPAIR = NCH // 2


_mesh = plsc.VectorSubcoreMesh(core_axis_name="c", subcore_axis_name="s")


@functools.partial(
    pl.kernel,
    mesh=_mesh,
    compiler_params=pltpu.CompilerParams(
        needs_layout_passes=False, use_tc_tiling_on_sc=False
    ),
    out_type=jax.ShapeDtypeStruct((B,), jnp.float32),
    scratch_types=[
        pltpu.VMEM((BPW,), jnp.int32),
        pltpu.VMEM((BPW,), jnp.int32),
        pltpu.VMEM((BPW * K,), jnp.int32),
        pltpu.VMEM((BPW * K,), jnp.int32),
        pltpu.VMEM((RPC, W), jnp.float32),
        pltpu.VMEM((RPC, W), jnp.float32),
        pltpu.VMEM((RPC, W), jnp.float32),
        pltpu.VMEM((RPC, W), jnp.float32),
        pltpu.VMEM((BPW,), jnp.float32),
        pltpu.SemaphoreType.DMA,
        pltpu.SemaphoreType.DMA,
    ],
)
def _pmf_sc(user_hbm, item_hbm, wu_v_hbm, wi_v_hbm, out_hbm,
            uidx, iidx, rlu, rli, ub0, ub1, ib0, ib1, oacc, sem0, sem1):
    wid = lax.axis_index("s") * NC + lax.axis_index("c")
    base = wid * BPW

    pltpu.sync_copy(user_hbm.at[pl.ds(base, BPW)], uidx)
    pltpu.sync_copy(item_hbm.at[pl.ds(base, BPW)], iidx)

    def build(g, carry):
        u = lax.shift_right_logical(uidx[pl.ds(g * L, L)], 4)
        v = lax.shift_right_logical(iidx[pl.ds(g * L, L)], 4)
        pos = (g // 2) * RPC + (g % 2) * L
        for k in range(K):
            rlu[pl.ds(pos + k * C, L)] = u + (k * RPK)
            rli[pl.ds(pos + k * C, L)] = v + (k * RPK)
        return carry

    lax.fori_loop(0, BPW // L, build, 0)

    def fire(c, bu, bi, sem):
        sl = pl.ds(c * RPC, RPC)
        pltpu.async_copy(wu_v_hbm.at[rlu.at[sl]], bu, sem)
        pltpu.async_copy(wi_v_hbm.at[rli.at[sl]], bi, sem)

    def drain(bu, bi, sem):
        pltpu.make_async_copy(wu_v_hbm.at[rlu.at[pl.ds(0, RPC)]], bu, sem).wait()
        pltpu.make_async_copy(wi_v_hbm.at[rli.at[pl.ds(0, RPC)]], bi, sem).wait()

    def compute(c, bu, bi):
        for g2 in range(C // L):
            isl = pl.ds(c * C + g2 * L, L)
            ulane = jnp.bitwise_and(uidx[isl], W - 1)
            ilane = jnp.bitwise_and(iidx[isl], W - 1)
            acc = jnp.zeros((L,), jnp.float32)
            for k in range(K):
                rows = k * C + g2 * L + lax.iota(jnp.int32, L)
                uval = plsc.load_gather(bu, [rows, ulane])
                ival = plsc.load_gather(bi, [rows, ilane])
                acc = acc + uval * ival
            oacc[isl] = acc

    fire(0, ub0, ib0, sem0)

    def pair(p, carry):
        c0 = p * 2
        fire(c0 + 1, ub1, ib1, sem1)
        drain(ub0, ib0, sem0)
        compute(c0, ub0, ib0)

        @pl.when(p < NPAIR - 1)
        def _():
            fire(c0 + 2, ub0, ib0, sem0)

        drain(ub1, ib1, sem1)
        compute(c0 + 1, ub1, ib1)
        return carry

    lax.fori_loop(0, NPAIR, pair, 0)

    pltpu.sync_copy(oacc, out_hbm.at[pl.ds(base, BPW)])


def kernel(user, item, W_user, W_item):
    wu_v = W_user.T.reshape(N_ROWS * K // W, W)
    wi_v = W_item.T.reshape(N_ROWS * K // W, W)
    return _pmf_sc(user, item, wu_v, wi_v)


# speedup vs baseline: 1.0027x; 1.0026x over previous
import functools

import jax
import jax.numpy as jnp
from jax import lax
from jax.experimental import pallas as pl
from jax.experimental.pallas import tpu as pltpu
from jax.experimental.pallas import tpu_sc as plsc

B = 16384
K = 32
N_ROWS = 1000000
NC = 2
NS = 16
NW = NC * NS
BPW = B // NW
L = 16
W = 16
RPK = N_ROWS // W
C = 32
RPC = C * K
NCH = BPW // C
NPAIR = NCH // 2


_mesh = plsc.VectorSubcoreMesh(core_axis_name="c", subcore_axis_name="s")


@functools.partial(
    pl.kernel,
    mesh=_mesh,
    compiler_params=pltpu.CompilerParams(
        needs_layout_passes=False, use_tc_tiling_on_sc=False
    ),
    out_type=jax.ShapeDtypeStruct((B,), jnp.float32),
    scratch_types=[
        pltpu.VMEM((BPW,), jnp.int32),
        pltpu.VMEM((BPW,), jnp.int32),
        pltpu.VMEM((BPW * K,), jnp.int32),
        pltpu.VMEM((BPW * K,), jnp.int32),
        pltpu.VMEM((RPC, W), jnp.float32),
        pltpu.VMEM((RPC, W), jnp.float32),
        pltpu.VMEM((RPC, W), jnp.float32),
        pltpu.VMEM((RPC, W), jnp.float32),
        pltpu.VMEM((BPW,), jnp.float32),
        pltpu.SemaphoreType.DMA,
        pltpu.SemaphoreType.DMA,
    ],
)
def _pmf_sc(user_hbm, item_hbm, wu_v_hbm, wi_v_hbm, out_hbm,
            uidx, iidx, rlu, rli, ub0, ub1, ib0, ib1, oacc, sem0, sem1):
    wid = lax.axis_index("s") * NC + lax.axis_index("c")
    base = wid * BPW

    pltpu.sync_copy(user_hbm.at[pl.ds(base, BPW)], uidx)
    pltpu.sync_copy(item_hbm.at[pl.ds(base, BPW)], iidx)

    def build(g, carry):
        u = lax.shift_right_logical(uidx[pl.ds(g * L, L)], 4)
        v = lax.shift_right_logical(iidx[pl.ds(g * L, L)], 4)
        pos = (g // 2) * RPC + (g % 2) * L
        for k in range(K):
            rlu[pl.ds(pos + k * C, L)] = u + (k * RPK)
            rli[pl.ds(pos + k * C, L)] = v + (k * RPK)
        return carry

    lax.fori_loop(0, BPW // L, build, 0)

    def fire(c, bu, bi, sem):
        for s in range(RPC // 128):
            sl = pl.ds(c * RPC + s * 128, 128)
            dsl = pl.ds(s * 128, 128)
            pltpu.async_copy(wu_v_hbm.at[rlu.at[sl]], bu.at[dsl, :], sem)
            pltpu.async_copy(wi_v_hbm.at[rli.at[sl]], bi.at[dsl, :], sem)

    def drain(bu, bi, sem):
        for s in range(RPC // 128):
            sl = pl.ds(s * 128, 128)
            pltpu.make_async_copy(
                wu_v_hbm.at[rlu.at[sl]], bu.at[sl, :], sem).wait()
            pltpu.make_async_copy(
                wi_v_hbm.at[rli.at[sl]], bi.at[sl, :], sem).wait()

    def compute(c, bu, bi):
        for g2 in range(C // L):
            isl = pl.ds(c * C + g2 * L, L)
            ulane = jnp.bitwise_and(uidx[isl], W - 1)
            ilane = jnp.bitwise_and(iidx[isl], W - 1)
            acc = jnp.zeros((L,), jnp.float32)
            for k in range(K):
                rows = k * C + g2 * L + lax.iota(jnp.int32, L)
                uval = plsc.load_gather(bu, [rows, ulane])
                ival = plsc.load_gather(bi, [rows, ilane])
                acc = acc + uval * ival
            oacc[isl] = acc

    fire(0, ub0, ib0, sem0)

    def pair(p, carry):
        c0 = p * 2
        fire(c0 + 1, ub1, ib1, sem1)
        drain(ub0, ib0, sem0)
        compute(c0, ub0, ib0)

        @pl.when(p < NPAIR - 1)
        def _():
            fire(c0 + 2, ub0, ib0, sem0)

        drain(ub1, ib1, sem1)
        compute(c0 + 1, ub1, ib1)
        return carry

    lax.fori_loop(0, NPAIR, pair, 0)

    pltpu.sync_copy(oacc, out_hbm.at[pl.ds(base, BPW)])


def kernel(user, item, W_user, W_item):
    wu_v = W_user.T.reshape(N_ROWS * K // W, W)
    wi_v = W_item.T.reshape(N_ROWS * K // W, W)
    return _pmf_sc(user, item, wu_v, wi_v)


# speedup vs baseline: 21.3265x; 21.2701x over previous
import functools

import jax
import jax.numpy as jnp
from jax import lax
from jax.experimental import pallas as pl
from jax.experimental.pallas import tpu as pltpu
from jax.experimental.pallas import tpu_sc as plsc

B = 16384
K = 32
KH = K // 2
N_ROWS = 1000000
NC = 2
NS = 16
NW = NC * NS
BPW = B // NW
C = 16
NCH = BPW // C
L = 16


_mesh = plsc.VectorSubcoreMesh(core_axis_name="c", subcore_axis_name="s")


@functools.partial(
    pl.kernel,
    mesh=_mesh,
    compiler_params=pltpu.CompilerParams(needs_layout_passes=False),
    out_type=jax.ShapeDtypeStruct((B,), jnp.float32),
    scratch_types=[
        pltpu.VMEM((BPW,), jnp.int32),
        pltpu.VMEM((BPW,), jnp.int32),
        pltpu.VMEM((C, KH, 128), jnp.float32),
        pltpu.VMEM((C, KH, 128), jnp.float32),
        pltpu.VMEM((BPW,), jnp.float32),
        pltpu.SemaphoreType.DMA,
    ],
)
def _pmf_sc(user_hbm, item_hbm, wu_t_hbm, wi_t_hbm, out_hbm,
            uvec, ivec, ublk, iblk, oacc, sem):
    wid = lax.axis_index("s") * NC + lax.axis_index("c")
    base = wid * BPW

    pltpu.sync_copy(user_hbm.at[pl.ds(base, BPW)], uvec)
    pltpu.sync_copy(item_hbm.at[pl.ds(base, BPW)], ivec)

    def chunk(c, carry):
        uv = uvec[pl.ds(c * C, L)]
        iv = ivec[pl.ds(c * C, L)]
        ulane = jnp.bitwise_and(uv, 127)
        ilane = jnp.bitwise_and(iv, 127)
        ubase = lax.shift_left(lax.shift_right_logical(uv, 7), 7)
        ibase = lax.shift_left(lax.shift_right_logical(iv, 7), 7)
        sel = lax.iota(jnp.int32, L)
        zero = jnp.zeros((L,), jnp.int32)
        acc = jnp.zeros((L,), jnp.float32)

        for kh in range(K // KH):
            copies = []
            for i in range(C):
                ub = pl.multiple_of(jnp.sum(jnp.where(sel == i, ubase, zero)), 128)
                ib = pl.multiple_of(jnp.sum(jnp.where(sel == i, ibase, zero)), 128)
                copies.append(pltpu.async_copy(
                    wu_t_hbm.at[pl.ds(kh * KH, KH), pl.ds(ub, 128)],
                    ublk.at[i], sem))
                copies.append(pltpu.async_copy(
                    wi_t_hbm.at[pl.ds(kh * KH, KH), pl.ds(ib, 128)],
                    iblk.at[i], sem))
            for cp in copies:
                cp.wait()

            for kk in range(KH):
                kvec = jnp.full((L,), kk, jnp.int32)
                u = plsc.load_gather(ublk, [sel, kvec, ulane])
                v = plsc.load_gather(iblk, [sel, kvec, ilane])
                acc = acc + u * v

        oacc[pl.ds(c * C, L)] = acc
        return carry

    lax.fori_loop(0, NCH, chunk, 0)

    pltpu.sync_copy(oacc, out_hbm.at[pl.ds(base, BPW)])


def kernel(user, item, W_user, W_item):
    return _pmf_sc(user, item, W_user.T, W_item.T)
